# baseline (speedup 1.0000x reference)
import functools

import jax
import jax.numpy as jnp
from jax import lax
from jax.experimental import pallas as pl
from jax.experimental.pallas import tpu as pltpu
from jax.experimental.pallas import tpu_sc as plsc

_B, _H, _W, _C = 2, 384, 384, 96
_P = _B * _H * _W
_NC, _NS = 2, 16
_NW = _NC * _NS
_PPW = _P // _NW
_CH = 128
_NCHUNK = _PPW // _CH
_CG = _C // 16


def _sc_body(feat_hbm, gy_hbm, gx_hbm, out_hbm,
             gy_v, gx_v,
             i00_v, i01_v, i10_v, i11_v,
             w00_v, w01_v, w10_v, w11_v,
             r00_v, r01_v, r10_v, r11_v,
             out_v, gsem0, gsem1, osem0, osem1):
    wid = lax.axis_index("s") * _NC + lax.axis_index("c")
    pix0 = wid * _PPW
    bbase = (pix0 // (_H * _W)) * (_H * _W)
    gsems = (gsem0, gsem1)
    osems = (osem0, osem1)

    def fire(jc, pr):
        base = pix0 + jc * _CH
        pltpu.sync_copy(gy_hbm.at[pl.ds(base, _CH)], gy_v)
        pltpu.sync_copy(gx_hbm.at[pl.ds(base, _CH)], gx_v)
        for i in range(_CH // 16):
            s = pl.ds(i * 16, 16)
            gr = (gy_v[s] + 1.0) * (_H * 0.5) - 0.5
            gc = (gx_v[s] + 1.0) * (_W * 0.5) - 0.5
            ty = gr.astype(jnp.int32)
            y0 = jnp.where(gr < ty.astype(jnp.float32), ty - 1, ty)
            y0f = y0.astype(jnp.float32)
            tx = gc.astype(jnp.int32)
            x0 = jnp.where(gc < tx.astype(jnp.float32), tx - 1, tx)
            x0f = x0.astype(jnp.float32)
            wy1 = gr - y0f
            wy0 = (y0f + 1.0) - gr
            wx1 = gc - x0f
            wx0 = (x0f + 1.0) - gc
            y1 = y0 + 1
            x1 = x0 + 1
            z = jnp.zeros_like(wy0)
            wy0 = jnp.where((y0 >= 0) & (y0 < _H), wy0, z)
            wy1 = jnp.where((y1 >= 0) & (y1 < _H), wy1, z)
            wx0 = jnp.where((x0 >= 0) & (x0 < _W), wx0, z)
            wx1 = jnp.where((x1 >= 0) & (x1 < _W), wx1, z)
            y0c = jnp.clip(y0, 0, _H - 1)
            y1c = jnp.clip(y1, 0, _H - 1)
            x0c = jnp.clip(x0, 0, _W - 1)
            x1c = jnp.clip(x1, 0, _W - 1)
            r0 = bbase + y0c * _W
            r1 = bbase + y1c * _W
            i00_v[pr, s] = r0 + x0c
            i01_v[pr, s] = r0 + x1c
            i10_v[pr, s] = r1 + x0c
            i11_v[pr, s] = r1 + x1c
            w00_v[pr, s] = wy0 * wx0
            w01_v[pr, s] = wy0 * wx1
            w10_v[pr, s] = wy1 * wx0
            w11_v[pr, s] = wy1 * wx1
        sem = gsems[pr]
        pltpu.async_copy(feat_hbm.at[i00_v.at[pr]], r00_v.at[pr], sem)
        pltpu.async_copy(feat_hbm.at[i01_v.at[pr]], r01_v.at[pr], sem)
        pltpu.async_copy(feat_hbm.at[i10_v.at[pr]], r10_v.at[pr], sem)
        pltpu.async_copy(feat_hbm.at[i11_v.at[pr]], r11_v.at[pr], sem)

    def drain_gather(pr):
        sem = gsems[pr]
        pltpu.make_async_copy(feat_hbm.at[i00_v.at[pr]], r00_v.at[pr], sem).wait()
        pltpu.make_async_copy(feat_hbm.at[i01_v.at[pr]], r01_v.at[pr], sem).wait()
        pltpu.make_async_copy(feat_hbm.at[i10_v.at[pr]], r10_v.at[pr], sem).wait()
        pltpu.make_async_copy(feat_hbm.at[i11_v.at[pr]], r11_v.at[pr], sem).wait()

    def combine(pr):
        @plsc.parallel_loop(0, _CH, unroll=2)
        def pix_body(p):
            pidx = jnp.full((16,), p, jnp.int32)
            pz = jnp.zeros((16,), jnp.int32)
            w00 = plsc.load_gather(w00_v, [pz + pr, pidx])
            w01 = plsc.load_gather(w01_v, [pz + pr, pidx])
            w10 = plsc.load_gather(w10_v, [pz + pr, pidx])
            w11 = plsc.load_gather(w11_v, [pz + pr, pidx])
            for cg in range(_CG):
                cs = pl.ds(cg * 16, 16)
                out_v[pr, pl.ds(p * _C + cg * 16, 16)] = (
                    w00 * r00_v[pr, p, cs] + w01 * r01_v[pr, p, cs]
                    + w10 * r10_v[pr, p, cs] + w11 * r11_v[pr, p, cs])

    def fire_out(jc, pr):
        base = (pix0 + jc * _CH) * _C
        pltpu.async_copy(out_v.at[pr], out_hbm.at[pl.ds(base, _CH * _C)],
                         osems[pr])

    def drain_out(jc, pr):
        base = (pix0 + jc * _CH) * _C
        pltpu.make_async_copy(out_v.at[pr],
                              out_hbm.at[pl.ds(base, _CH * _C)],
                              osems[pr]).wait()

    fire(0, 0)

    def pair_body(j2, carry):
        ja = 2 * j2
        jb = ja + 1
        fire(jb, 1)
        drain_gather(0)

        @pl.when(j2 > 0)
        def _():
            drain_out(ja - 2, 0)

        combine(0)
        fire_out(ja, 0)

        @pl.when(jb + 1 < _NCHUNK)
        def _():
            fire(jb + 1, 0)

        drain_gather(1)

        @pl.when(j2 > 0)
        def _():
            drain_out(jb - 2, 1)

        combine(1)
        fire_out(jb, 1)
        return carry

    lax.fori_loop(0, _NCHUNK // 2, pair_body, 0)
    drain_out(_NCHUNK - 2, 0)
    drain_out(_NCHUNK - 1, 1)


_grid_sample_call = functools.partial(
    pl.kernel,
    out_type=jax.ShapeDtypeStruct((_P * _C,), jnp.float32),
    mesh=plsc.VectorSubcoreMesh(core_axis_name="c", subcore_axis_name="s",
                                num_cores=_NC, num_subcores=_NS),
    compiler_params=pltpu.CompilerParams(needs_layout_passes=False,
                                         use_tc_tiling_on_sc=False),
    scratch_types=[
        pltpu.VMEM((_CH,), jnp.float32),
        pltpu.VMEM((_CH,), jnp.float32),
        pltpu.VMEM((2, _CH), jnp.int32),
        pltpu.VMEM((2, _CH), jnp.int32),
        pltpu.VMEM((2, _CH), jnp.int32),
        pltpu.VMEM((2, _CH), jnp.int32),
        pltpu.VMEM((2, _CH), jnp.float32),
        pltpu.VMEM((2, _CH), jnp.float32),
        pltpu.VMEM((2, _CH), jnp.float32),
        pltpu.VMEM((2, _CH), jnp.float32),
        pltpu.VMEM((2, _CH, _C), jnp.float32),
        pltpu.VMEM((2, _CH, _C), jnp.float32),
        pltpu.VMEM((2, _CH, _C), jnp.float32),
        pltpu.VMEM((2, _CH, _C), jnp.float32),
        pltpu.VMEM((2, _CH * _C), jnp.float32),
        pltpu.SemaphoreType.DMA,
        pltpu.SemaphoreType.DMA,
        pltpu.SemaphoreType.DMA,
        pltpu.SemaphoreType.DMA,
    ],
)(_sc_body)


def kernel(features, grid):
    B, H, W, C = features.shape
    feat = features.reshape(B * H * W, C)
    gx = grid[..., 0].reshape(-1)
    gy = grid[..., 1].reshape(-1)
    out = _grid_sample_call(feat, gy, gx)
    return out.reshape(B, H, W, C)


# speedup vs baseline: 1.2180x; 1.2180x over previous
import functools

import jax
import jax.numpy as jnp
from jax import lax
from jax.experimental import pallas as pl
from jax.experimental.pallas import tpu as pltpu
from jax.experimental.pallas import tpu_sc as plsc

_B, _H, _W, _C = 2, 384, 384, 96
_P = _B * _H * _W
_NC, _NS = 2, 16
_NW = _NC * _NS
_PPW = _P // _NW
_CH = 128
_NCHUNK = _PPW // _CH
_CG = _C // 16


def _sc_body(feat_hbm, gy_hbm, gx_hbm, out_hbm,
             gy_v, gx_v,
             i00_v, i01_v, i10_v, i11_v,
             w00_v, w01_v, w10_v, w11_v,
             r00_v, r01_v, r10_v, r11_v,
             out_v, gsem0, gsem1, osem0, osem1):
    wid = lax.axis_index("s") * _NC + lax.axis_index("c")
    pix0 = wid * _PPW
    bbase = (pix0 // (_H * _W)) * (_H * _W)
    gsems = (gsem0, gsem1)
    osems = (osem0, osem1)

    def fire(jc, pr):
        base = pix0 + jc * _CH
        pltpu.sync_copy(gy_hbm.at[pl.ds(base, _CH)], gy_v)
        pltpu.sync_copy(gx_hbm.at[pl.ds(base, _CH)], gx_v)
        for i in range(_CH // 16):
            s = pl.ds(i * 16, 16)
            gr = (gy_v[s] + 1.0) * (_H * 0.5) - 0.5
            gc = (gx_v[s] + 1.0) * (_W * 0.5) - 0.5
            ty = gr.astype(jnp.int32)
            y0 = jnp.where(gr < ty.astype(jnp.float32), ty - 1, ty)
            y0f = y0.astype(jnp.float32)
            tx = gc.astype(jnp.int32)
            x0 = jnp.where(gc < tx.astype(jnp.float32), tx - 1, tx)
            x0f = x0.astype(jnp.float32)
            wy1 = gr - y0f
            wy0 = (y0f + 1.0) - gr
            wx1 = gc - x0f
            wx0 = (x0f + 1.0) - gc
            y1 = y0 + 1
            x1 = x0 + 1
            z = jnp.zeros_like(wy0)
            wy0 = jnp.where((y0 >= 0) & (y0 < _H), wy0, z)
            wy1 = jnp.where((y1 >= 0) & (y1 < _H), wy1, z)
            wx0 = jnp.where((x0 >= 0) & (x0 < _W), wx0, z)
            wx1 = jnp.where((x1 >= 0) & (x1 < _W), wx1, z)
            y0c = jnp.clip(y0, 0, _H - 1)
            y1c = jnp.clip(y1, 0, _H - 1)
            x0c = jnp.clip(x0, 0, _W - 1)
            x1c = jnp.clip(x1, 0, _W - 1)
            r0 = bbase + y0c * _W
            r1 = bbase + y1c * _W
            i00_v[pr, s] = r0 + x0c
            i01_v[pr, s] = r0 + x1c
            i10_v[pr, s] = r1 + x0c
            i11_v[pr, s] = r1 + x1c
            w00_v[pr, s] = wy0 * wx0
            w01_v[pr, s] = wy0 * wx1
            w10_v[pr, s] = wy1 * wx0
            w11_v[pr, s] = wy1 * wx1
        sem = gsems[pr]
        pltpu.async_copy(feat_hbm.at[i00_v.at[pr]], r00_v.at[pr], sem)
        pltpu.async_copy(feat_hbm.at[i01_v.at[pr]], r01_v.at[pr], sem)
        pltpu.async_copy(feat_hbm.at[i10_v.at[pr]], r10_v.at[pr], sem)
        pltpu.async_copy(feat_hbm.at[i11_v.at[pr]], r11_v.at[pr], sem)

    def drain_gather(pr):
        sem = gsems[pr]
        pltpu.make_async_copy(feat_hbm.at[i00_v.at[pr]], r00_v.at[pr], sem).wait()
        pltpu.make_async_copy(feat_hbm.at[i01_v.at[pr]], r01_v.at[pr], sem).wait()
        pltpu.make_async_copy(feat_hbm.at[i10_v.at[pr]], r10_v.at[pr], sem).wait()
        pltpu.make_async_copy(feat_hbm.at[i11_v.at[pr]], r11_v.at[pr], sem).wait()

    def combine(pr):
        @plsc.parallel_loop(0, _CH, unroll=2)
        def pix_body(p):
            pidx = jnp.full((16,), p, jnp.int32)
            pz = jnp.zeros((16,), jnp.int32)
            w00 = plsc.load_gather(w00_v, [pz + pr, pidx])
            w01 = plsc.load_gather(w01_v, [pz + pr, pidx])
            w10 = plsc.load_gather(w10_v, [pz + pr, pidx])
            w11 = plsc.load_gather(w11_v, [pz + pr, pidx])
            for cg in range(_CG):
                cs = pl.ds(cg * 16, 16)
                out_v[pr, p, cs] = (
                    w00 * r00_v[pr, p, cs] + w01 * r01_v[pr, p, cs]
                    + w10 * r10_v[pr, p, cs] + w11 * r11_v[pr, p, cs])

    def fire_out(jc, pr):
        base = pix0 + jc * _CH
        pltpu.async_copy(out_v.at[pr],
                         out_hbm.at[pl.ds(base, _CH), pl.ds(0, _C)],
                         osems[pr])

    def drain_out(jc, pr):
        base = pix0 + jc * _CH
        pltpu.make_async_copy(out_v.at[pr],
                              out_hbm.at[pl.ds(base, _CH), pl.ds(0, _C)],
                              osems[pr]).wait()

    fire(0, 0)

    def pair_body(j2, carry):
        ja = 2 * j2
        jb = ja + 1
        fire(jb, 1)
        drain_gather(0)

        @pl.when(j2 > 0)
        def _():
            drain_out(ja - 2, 0)

        combine(0)
        fire_out(ja, 0)

        @pl.when(jb + 1 < _NCHUNK)
        def _():
            fire(jb + 1, 0)

        drain_gather(1)

        @pl.when(j2 > 0)
        def _():
            drain_out(jb - 2, 1)

        combine(1)
        fire_out(jb, 1)
        return carry

    lax.fori_loop(0, _NCHUNK // 2, pair_body, 0)
    drain_out(_NCHUNK - 2, 0)
    drain_out(_NCHUNK - 1, 1)


_grid_sample_call = functools.partial(
    pl.kernel,
    out_type=jax.ShapeDtypeStruct((_P, 128), jnp.float32),
    mesh=plsc.VectorSubcoreMesh(core_axis_name="c", subcore_axis_name="s",
                                num_cores=_NC, num_subcores=_NS),
    compiler_params=pltpu.CompilerParams(needs_layout_passes=False,
                                         use_tc_tiling_on_sc=False),
    scratch_types=[
        pltpu.VMEM((_CH,), jnp.float32),
        pltpu.VMEM((_CH,), jnp.float32),
        pltpu.VMEM((2, _CH), jnp.int32),
        pltpu.VMEM((2, _CH), jnp.int32),
        pltpu.VMEM((2, _CH), jnp.int32),
        pltpu.VMEM((2, _CH), jnp.int32),
        pltpu.VMEM((2, _CH), jnp.float32),
        pltpu.VMEM((2, _CH), jnp.float32),
        pltpu.VMEM((2, _CH), jnp.float32),
        pltpu.VMEM((2, _CH), jnp.float32),
        pltpu.VMEM((2, _CH, _C), jnp.float32),
        pltpu.VMEM((2, _CH, _C), jnp.float32),
        pltpu.VMEM((2, _CH, _C), jnp.float32),
        pltpu.VMEM((2, _CH, _C), jnp.float32),
        pltpu.VMEM((2, _CH, _C), jnp.float32),
        pltpu.SemaphoreType.DMA,
        pltpu.SemaphoreType.DMA,
        pltpu.SemaphoreType.DMA,
        pltpu.SemaphoreType.DMA,
    ],
)(_sc_body)


def kernel(features, grid):
    B, H, W, C = features.shape
    feat = features.reshape(B * H * W, C)
    gx = grid[..., 0].reshape(-1)
    gy = grid[..., 1].reshape(-1)
    out = _grid_sample_call(feat, gy, gx)
    return out[:, :C].reshape(B, H, W, C)
